# baseline (speedup 1.0000x reference)
import jax
import jax.numpy as jnp
from jax import lax
from jax.experimental import pallas as pl
from jax.experimental.pallas import tpu as pltpu
from jax.experimental.pallas import tpu_sc as plsc

N_NODES = 10000
N_EDGES = 320000
C = 128
NC = 2
NS = 16
NW = NC * NS
K = 80
NCHUNK = N_EDGES // K
NCHUNK_CORE = NCHUNK // NC
NCH_TILE = NCHUNK_CORE // NS
NR = 624
NR_LAST_EXTRA = N_NODES - NS * NR
SHIFT = 14


def _transpose_body(x_ref, o_ref):
    o_ref[...] = x_ref[...].T


def _transpose(x2d):
    return pl.pallas_call(
        _transpose_body,
        out_shape=jax.ShapeDtypeStruct((N_NODES, C), jnp.float32),
    )(x2d)


def _pack_body(i_ref, j_ref, o_ref):
    o_ref[...] = (i_ref[...] << SHIFT) | j_ref[...]


def _pack(iInd, jInd):
    return pl.pallas_call(
        _pack_body,
        out_shape=jax.ShapeDtypeStruct((NCHUNK, K), jnp.int32),
    )(iInd.reshape(NCHUNK, K), jInd.reshape(NCHUNK, K)).reshape(
        NC, N_EDGES // NC)


def _edge_kernel_body(xT, Wh, pk, om_hbm, osum_hbm, w_v, s_v, om,
                      ii, jj, rows, gsem):
    cid = lax.axis_index("c")
    sid = lax.axis_index("s")

    zero16 = jnp.zeros((16,), jnp.float32)

    def zrow(r, carry):
        for v in range(C // 16):
            rows[0, r, pl.ds(v * 16, 16)] = zero16
        return carry

    lax.fori_loop(0, K, zrow, 0)

    def zs(r, carry):
        s_v[pl.ds(r * 16, 16)] = zero16
        return carry

    lax.fori_loop(0, N_NODES // 16, zs, 0)

    nbase = sid * NR
    def zacc(t, carry):
        pltpu.sync_copy(rows.at[0, pl.ds(0, 78)],
                        om.at[pl.ds(nbase + t * 78, 78)])
        return carry
    lax.fori_loop(0, NR // 78, zacc, 0)

    @pl.when(sid == NS - 1)
    def _zero_tail():
        pltpu.sync_copy(rows.at[0, pl.ds(0, NR_LAST_EXTRA)],
                        om.at[pl.ds(NS * NR, NR_LAST_EXTRA)])

    pltpu.sync_copy(Wh, w_v)
    plsc.subcore_barrier()

    def body(t, carry):
        b = t % 2

        @pl.when(t < NCH_TILE)
        def _issue():
            base = (sid + t * NS) * K
            pltpu.sync_copy(pk.at[cid, pl.ds(base, K)], jj.at[b])

            def unpack(g, c2):
                v = jj[b, pl.ds(g * 16, 16)]
                ii[b, pl.ds(g * 16, 16)] = v >> SHIFT
                jj[b, pl.ds(g * 16, 16)] = v & ((1 << SHIFT) - 1)
                return c2

            lax.fori_loop(0, K // 16, unpack, 0)
            pltpu.async_copy(xT.at[jj.at[b]], rows.at[b], gsem.at[b])

        @pl.when(t >= 1)
        def _process():
            tb = (t - 1) % 2
            pltpu.make_async_copy(xT.at[pl.ds(0, K)], rows.at[tb],
                                  gsem.at[tb]).wait()

            def grp(g, c2):
                vi = ii[tb, pl.ds(g * 16, 16)]
                vj = jj[tb, pl.ds(g * 16, 16)]
                wi = plsc.load_gather(w_v, [vi])
                wj = plsc.load_gather(w_v, [vj])
                cv = wi * (wi + wj)
                plsc.addupdate_scatter(s_v, [vi], cv)
                for k in range(16):
                    e = g * 16 + k
                    cs = cv[k]
                    for v in range(C // 16):
                        rows[tb, e, pl.ds(v * 16, 16)] = (
                            rows[tb, e, pl.ds(v * 16, 16)] * cs)
                return c2

            lax.fori_loop(0, K // 16, grp, 0)
            pltpu.sync_copy(rows.at[tb], om.at[ii.at[tb]], add=True)

        return carry

    lax.fori_loop(0, NCH_TILE + 1, body, 0)
    plsc.subcore_barrier()

    pltpu.sync_copy(om.at[pl.ds(nbase, NR)],
                    om_hbm.at[cid, pl.ds(nbase, NR)])

    @pl.when(sid == NS - 1)
    def _tail():
        pltpu.sync_copy(om.at[pl.ds(NS * NR, NR_LAST_EXTRA)],
                        om_hbm.at[cid, pl.ds(NS * NR, NR_LAST_EXTRA)])

    wid = cid * NS + sid
    pltpu.sync_copy(s_v, osum_hbm.at[pl.ds(wid * N_NODES, N_NODES)])


def _edge_scatter(xT, W, packed):
    mesh = plsc.VectorSubcoreMesh(core_axis_name="c", subcore_axis_name="s",
                                  num_cores=NC, num_subcores=NS)
    f = pl.kernel(
        _edge_kernel_body,
        out_type=(jax.ShapeDtypeStruct((NC, N_NODES, C), jnp.float32),
                  jax.ShapeDtypeStruct((NW * N_NODES,), jnp.float32)),
        mesh=mesh,
        compiler_params=pltpu.CompilerParams(needs_layout_passes=False,
                                             use_tc_tiling_on_sc=False),
        scratch_types=[
            pltpu.VMEM((N_NODES,), jnp.float32),
            pltpu.VMEM((N_NODES,), jnp.float32),
            pltpu.VMEM_SHARED((N_NODES, C), jnp.float32),
            pltpu.VMEM((2, K), jnp.int32),
            pltpu.VMEM((2, K), jnp.int32),
            pltpu.VMEM((2, K, C), jnp.float32),
            pltpu.SemaphoreType.DMA((2,)),
        ],
    )
    return f(xT, W, packed)


def _combine_body(x_ref, am_ref, as_ref, o_ref):
    s = jnp.sum(as_ref[...], axis=0, keepdims=True)
    a = am_ref[0] + am_ref[1]
    o_ref[...] = x_ref[...] * s - a.T


def _combine(x2d, om, osum):
    return pl.pallas_call(
        _combine_body,
        out_shape=jax.ShapeDtypeStruct((C, N_NODES), jnp.float32),
    )(x2d, om, osum.reshape(NW, N_NODES))


def kernel(x, W, iInd, jInd):
    x2d = x[0]
    xT = _transpose(x2d)
    packed = _pack(iInd.astype(jnp.int32), jInd.astype(jnp.int32))
    om, osum = _edge_scatter(xT, W, packed)
    out2d = _combine(x2d, om, osum)
    return out2d[None]


# speedup vs baseline: 2.2889x; 2.2889x over previous
import jax
import jax.numpy as jnp
from jax import lax
from jax.experimental import pallas as pl
from jax.experimental.pallas import tpu as pltpu
from jax.experimental.pallas import tpu_sc as plsc

N_NODES = 10000
N_EDGES = 320000
C = 128
NC = 2
NS = 16
NW = NC * NS
K = 80
NCHUNK = N_EDGES // K
NCHUNK_CORE = NCHUNK // NC
NCH_TILE = NCHUNK_CORE // NS
NR = 624
NR_LAST_EXTRA = N_NODES - NS * NR
SHIFT = 14


def _transpose_body(x_ref, o_ref):
    o_ref[...] = x_ref[...].T


def _transpose(x2d):
    return pl.pallas_call(
        _transpose_body,
        out_shape=jax.ShapeDtypeStruct((N_NODES, C), jnp.float32),
    )(x2d)


def _pack_body(i_ref, j_ref, o_ref):
    o_ref[...] = (i_ref[...] << SHIFT) | j_ref[...]


def _pack(iInd, jInd):
    return pl.pallas_call(
        _pack_body,
        out_shape=jax.ShapeDtypeStruct((NCHUNK, K), jnp.int32),
    )(iInd.reshape(NCHUNK, K), jInd.reshape(NCHUNK, K)).reshape(
        NC, N_EDGES // NC)


def _edge_kernel_body(xT, Wh, pk, om_hbm, osum_hbm, w_v, s_v, om,
                      ii0, ii1, jj0, jj1, r0, r1, g0, g1):
    cid = lax.axis_index("c")
    sid = lax.axis_index("s")

    zero16 = jnp.zeros((16,), jnp.float32)

    def zrow(r, carry):
        for v in range(C // 16):
            r0[r, pl.ds(v * 16, 16)] = zero16
        return carry

    lax.fori_loop(0, K, zrow, 0)

    def zs(r, carry):
        s_v[pl.ds(r * 16, 16)] = zero16
        return carry

    lax.fori_loop(0, N_NODES // 16, zs, 0)

    nbase = sid * NR
    def zacc(t, carry):
        pltpu.sync_copy(r0.at[pl.ds(0, 78)],
                        om.at[pl.ds(nbase + t * 78, 78)])
        return carry
    lax.fori_loop(0, NR // 78, zacc, 0)

    @pl.when(sid == NS - 1)
    def _zero_tail():
        pltpu.sync_copy(r0.at[pl.ds(0, NR_LAST_EXTRA)],
                        om.at[pl.ds(NS * NR, NR_LAST_EXTRA)])

    pltpu.sync_copy(Wh, w_v)
    plsc.subcore_barrier()

    def issue(t, iib, jjb, rb, gb):
        base = (sid + t * NS) * K
        pltpu.sync_copy(pk.at[cid, pl.ds(base, K)], jjb)

        def unpack(g, c2):
            v = jjb[pl.ds(g * 16, 16)]
            iib[pl.ds(g * 16, 16)] = v >> SHIFT
            jjb[pl.ds(g * 16, 16)] = v & ((1 << SHIFT) - 1)
            return c2

        lax.fori_loop(0, K // 16, unpack, 0)
        pltpu.async_copy(xT.at[jjb], rb, gb)

    def process(iib, jjb, rb, gb):
        pltpu.make_async_copy(xT.at[pl.ds(0, K)], rb, gb).wait()

        def grp(g, c2):
            vi = iib[pl.ds(g * 16, 16)]
            vj = jjb[pl.ds(g * 16, 16)]
            wi = plsc.load_gather(w_v, [vi])
            wj = plsc.load_gather(w_v, [vj])
            cv = wi * (wi + wj)
            plsc.addupdate_scatter(s_v, [vi], cv)
            for k in range(16):
                e = g * 16 + k
                cs = cv[k]
                for v in range(C // 16):
                    rb[e, pl.ds(v * 16, 16)] = rb[e, pl.ds(v * 16, 16)] * cs
            return c2

        lax.fori_loop(0, K // 16, grp, 0)
        pltpu.sync_copy(rb, om.at[iib], add=True)

    issue(0, ii0, jj0, r0, g0)

    def pair_body(tp, carry):
        t0 = tp * 2

        @pl.when(t0 + 1 < NCH_TILE)
        def _i1():
            issue(t0 + 1, ii1, jj1, r1, g1)

        process(ii0, jj0, r0, g0)

        @pl.when(t0 + 2 < NCH_TILE)
        def _i2():
            issue(t0 + 2, ii0, jj0, r0, g0)

        @pl.when(t0 + 1 < NCH_TILE)
        def _p2():
            process(ii1, jj1, r1, g1)

        return carry

    lax.fori_loop(0, (NCH_TILE + 1) // 2, pair_body, 0)
    plsc.subcore_barrier()

    pltpu.sync_copy(om.at[pl.ds(nbase, NR)],
                    om_hbm.at[cid, pl.ds(nbase, NR)])

    @pl.when(sid == NS - 1)
    def _tail():
        pltpu.sync_copy(om.at[pl.ds(NS * NR, NR_LAST_EXTRA)],
                        om_hbm.at[cid, pl.ds(NS * NR, NR_LAST_EXTRA)])

    wid = cid * NS + sid
    pltpu.sync_copy(s_v, osum_hbm.at[pl.ds(wid * N_NODES, N_NODES)])


def _edge_scatter(xT, W, packed):
    mesh = plsc.VectorSubcoreMesh(core_axis_name="c", subcore_axis_name="s",
                                  num_cores=NC, num_subcores=NS)
    f = pl.kernel(
        _edge_kernel_body,
        out_type=(jax.ShapeDtypeStruct((NC, N_NODES, C), jnp.float32),
                  jax.ShapeDtypeStruct((NW * N_NODES,), jnp.float32)),
        mesh=mesh,
        compiler_params=pltpu.CompilerParams(needs_layout_passes=False,
                                             use_tc_tiling_on_sc=False),
        scratch_types=[
            pltpu.VMEM((N_NODES,), jnp.float32),
            pltpu.VMEM((N_NODES,), jnp.float32),
            pltpu.VMEM_SHARED((N_NODES, C), jnp.float32),
            pltpu.VMEM((K,), jnp.int32),
            pltpu.VMEM((K,), jnp.int32),
            pltpu.VMEM((K,), jnp.int32),
            pltpu.VMEM((K,), jnp.int32),
            pltpu.VMEM((K, C), jnp.float32),
            pltpu.VMEM((K, C), jnp.float32),
            pltpu.SemaphoreType.DMA,
            pltpu.SemaphoreType.DMA,
        ],
    )
    return f(xT, W, packed)


def _combine_body(x_ref, am_ref, as_ref, o_ref):
    s = jnp.sum(as_ref[...], axis=0, keepdims=True)
    a = am_ref[0] + am_ref[1]
    o_ref[...] = x_ref[...] * s - a.T


def _combine(x2d, om, osum):
    return pl.pallas_call(
        _combine_body,
        out_shape=jax.ShapeDtypeStruct((C, N_NODES), jnp.float32),
    )(x2d, om, osum.reshape(NW, N_NODES))


def kernel(x, W, iInd, jInd):
    x2d = x[0]
    xT = _transpose(x2d)
    packed = _pack(iInd.astype(jnp.int32), jInd.astype(jnp.int32))
    om, osum = _edge_scatter(xT, W, packed)
    out2d = _combine(x2d, om, osum)
    return out2d[None]


# speedup vs baseline: 2.7308x; 1.1931x over previous
import jax
import jax.numpy as jnp
from jax import lax
from jax.experimental import pallas as pl
from jax.experimental.pallas import tpu as pltpu
from jax.experimental.pallas import tpu_sc as plsc

N_NODES = 10000
N_EDGES = 320000
C = 128
NC = 2
NS = 16
NW = NC * NS
K = 80
NCHUNK = N_EDGES // K
NCHUNK_CORE = NCHUNK // NC
NCH_TILE = NCHUNK_CORE // NS
NR = 624
NR_LAST_EXTRA = N_NODES - NS * NR
SHIFT = 14


def _transpose_body(x_ref, o_ref):
    o_ref[...] = x_ref[...].T


def _transpose(x2d):
    return pl.pallas_call(
        _transpose_body,
        out_shape=jax.ShapeDtypeStruct((N_NODES, C), jnp.float32),
    )(x2d)


def _pack_body(i_ref, j_ref, o_ref):
    o_ref[...] = (i_ref[...] << SHIFT) | j_ref[...]


def _pack(iInd, jInd):
    return pl.pallas_call(
        _pack_body,
        out_shape=jax.ShapeDtypeStruct((NCHUNK, K), jnp.int32),
    )(iInd.reshape(NCHUNK, K), jInd.reshape(NCHUNK, K)).reshape(
        NC, N_EDGES // NC)


def _edge_kernel_body(xT, Wh, pk, om_hbm, osum_hbm, w_v, s_v, om,
                      ii0, ii1, jj0, jj1, r0, r1, pk0, pk1,
                      g0, g1, s0, s1, p0, p1):
    cid = lax.axis_index("c")
    sid = lax.axis_index("s")

    zero16 = jnp.zeros((16,), jnp.float32)

    def zrow(r, carry):
        for v in range(C // 16):
            r0[r, pl.ds(v * 16, 16)] = zero16
        return carry

    lax.fori_loop(0, K, zrow, 0)

    def zs(r, carry):
        s_v[pl.ds(r * 16, 16)] = zero16
        return carry

    lax.fori_loop(0, N_NODES // 16, zs, 0)

    nbase = sid * NR
    def zacc(t, carry):
        pltpu.sync_copy(r0.at[pl.ds(0, 78)],
                        om.at[pl.ds(nbase + t * 78, 78)])
        return carry
    lax.fori_loop(0, NR // 78, zacc, 0)

    @pl.when(sid == NS - 1)
    def _zero_tail():
        pltpu.sync_copy(r0.at[pl.ds(0, NR_LAST_EXTRA)],
                        om.at[pl.ds(NS * NR, NR_LAST_EXTRA)])

    pltpu.sync_copy(Wh, w_v)
    plsc.subcore_barrier()

    def pk_fetch(t, pkb, pb):
        base = (sid + t * NS) * K
        pltpu.async_copy(pk.at[cid, pl.ds(base, K)], pkb, pb)

    def issue(t, iib, jjb, rb, gb, pkb, pb):
        pltpu.make_async_copy(pk.at[cid, pl.ds(0, K)], pkb, pb).wait()

        def unpack(g, c2):
            v = pkb[pl.ds(g * 16, 16)]
            iib[pl.ds(g * 16, 16)] = v >> SHIFT
            jjb[pl.ds(g * 16, 16)] = v & ((1 << SHIFT) - 1)
            return c2

        lax.fori_loop(0, K // 16, unpack, 0)
        pltpu.async_copy(xT.at[jjb], rb, gb)

        @pl.when(t + 2 < NCH_TILE)
        def _prefetch():
            pk_fetch(t + 2, pkb, pb)

    def process(iib, jjb, rb, gb, sb):
        pltpu.make_async_copy(xT.at[pl.ds(0, K)], rb, gb).wait()

        def grp(g, c2):
            vi = iib[pl.ds(g * 16, 16)]
            vj = jjb[pl.ds(g * 16, 16)]
            wi = plsc.load_gather(w_v, [vi])
            wj = plsc.load_gather(w_v, [vj])
            cv = wi * (wi + wj)
            plsc.addupdate_scatter(s_v, [vi], cv)
            for k in range(16):
                e = g * 16 + k
                cs = cv[k]
                for v in range(C // 16):
                    rb[e, pl.ds(v * 16, 16)] = rb[e, pl.ds(v * 16, 16)] * cs
            return c2

        lax.fori_loop(0, K // 16, grp, 0)
        pltpu.async_copy(rb, om.at[iib], sb, add=True)

    def drain_scatter(iib, rb, sb):
        pltpu.make_async_copy(rb, om.at[iib], sb).wait()

    pk_fetch(0, pk0, p0)
    pk_fetch(1, pk1, p1)
    issue(0, ii0, jj0, r0, g0, pk0, p0)

    def pair_body(tp, carry):
        t0 = tp * 2

        @pl.when(t0 + 1 < NCH_TILE)
        def _i1():
            @pl.when(t0 >= 1)
            def _d1():
                drain_scatter(ii1, r1, s1)
            issue(t0 + 1, ii1, jj1, r1, g1, pk1, p1)

        process(ii0, jj0, r0, g0, s0)

        @pl.when(t0 + 2 < NCH_TILE)
        def _i2():
            drain_scatter(ii0, r0, s0)
            issue(t0 + 2, ii0, jj0, r0, g0, pk0, p0)

        @pl.when(t0 + 1 < NCH_TILE)
        def _p2():
            process(ii1, jj1, r1, g1, s1)

        return carry

    lax.fori_loop(0, (NCH_TILE + 1) // 2, pair_body, 0)
    drain_scatter(ii0, r0, s0)
    drain_scatter(ii1, r1, s1)
    plsc.subcore_barrier()

    pltpu.sync_copy(om.at[pl.ds(nbase, NR)],
                    om_hbm.at[cid, pl.ds(nbase, NR)])

    @pl.when(sid == NS - 1)
    def _tail():
        pltpu.sync_copy(om.at[pl.ds(NS * NR, NR_LAST_EXTRA)],
                        om_hbm.at[cid, pl.ds(NS * NR, NR_LAST_EXTRA)])

    wid = cid * NS + sid
    pltpu.sync_copy(s_v, osum_hbm.at[pl.ds(wid * N_NODES, N_NODES)])


def _edge_scatter(xT, W, packed):
    mesh = plsc.VectorSubcoreMesh(core_axis_name="c", subcore_axis_name="s",
                                  num_cores=NC, num_subcores=NS)
    f = pl.kernel(
        _edge_kernel_body,
        out_type=(jax.ShapeDtypeStruct((NC, N_NODES, C), jnp.float32),
                  jax.ShapeDtypeStruct((NW * N_NODES,), jnp.float32)),
        mesh=mesh,
        compiler_params=pltpu.CompilerParams(needs_layout_passes=False,
                                             use_tc_tiling_on_sc=False),
        scratch_types=[
            pltpu.VMEM((N_NODES,), jnp.float32),
            pltpu.VMEM((N_NODES,), jnp.float32),
            pltpu.VMEM_SHARED((N_NODES, C), jnp.float32),
            pltpu.VMEM((K,), jnp.int32),
            pltpu.VMEM((K,), jnp.int32),
            pltpu.VMEM((K,), jnp.int32),
            pltpu.VMEM((K,), jnp.int32),
            pltpu.VMEM((K, C), jnp.float32),
            pltpu.VMEM((K, C), jnp.float32),
            pltpu.VMEM((K,), jnp.int32),
            pltpu.VMEM((K,), jnp.int32),
            pltpu.SemaphoreType.DMA,
            pltpu.SemaphoreType.DMA,
            pltpu.SemaphoreType.DMA,
            pltpu.SemaphoreType.DMA,
            pltpu.SemaphoreType.DMA,
            pltpu.SemaphoreType.DMA,
        ],
    )
    return f(xT, W, packed)


def _combine_body(x_ref, am_ref, as_ref, o_ref):
    s = jnp.sum(as_ref[...], axis=0, keepdims=True)
    a = am_ref[0] + am_ref[1]
    o_ref[...] = x_ref[...] * s - a.T


def _combine(x2d, om, osum):
    return pl.pallas_call(
        _combine_body,
        out_shape=jax.ShapeDtypeStruct((C, N_NODES), jnp.float32),
    )(x2d, om, osum.reshape(NW, N_NODES))


def kernel(x, W, iInd, jInd):
    x2d = x[0]
    xT = _transpose(x2d)
    packed = _pack(iInd.astype(jnp.int32), jInd.astype(jnp.int32))
    om, osum = _edge_scatter(xT, W, packed)
    out2d = _combine(x2d, om, osum)
    return out2d[None]
